# Initial kernel scaffold; baseline (speedup 1.0000x reference)
import jax
import jax.numpy as jnp
from jax.experimental import pallas as pl


def kernel(x, table_0, table_1, table_2, table_3, W1, b1, W2, b2, W3, b3):
    raise NotImplementedError("write your pallas kernel here")


# speedup vs baseline: 3.9653x; 3.9653x over previous
import functools

import jax
import jax.numpy as jnp
from jax import lax
from jax.experimental import pallas as pl
from jax.experimental.pallas import tpu as pltpu
import jax.experimental.pallas.tpu_sc as plsc

B, L, NF = 4096, 50, 4
VOCAB, DIM = 100000, 32
OUT_DIM = NF * DIM + 3
N = B * L
LANES = 128
NROWS = N // LANES
NW = 32
ROWS_PER_W = NROWS // NW
TOK_PER_W = N // NW
GROUP = 5
NCHUNK = ROWS_PER_W // GROUP
CHUNK_TOK = GROUP * LANES


def _sc_gather_body(t0, t1, t2, t3, i0, i1, i2, i3, g_hbm, idx_v, buf_v, sem):
    tables = (t0, t1, t2, t3)
    idxs = (i0, i1, i2, i3)
    wid = lax.axis_index("s") * 2 + lax.axis_index("c")
    base_tok = wid * TOK_PER_W

    for f in range(NF):
        pltpu.sync_copy(idxs[f].at[wid], idx_v.at[f])

    def chunk(c, carry):
        copies = []
        for f in range(NF):
            for j in range(GROUP):
                r = c * GROUP + j
                copies.append(
                    pltpu.async_copy(
                        tables[f].at[idx_v.at[f, r]],
                        buf_v.at[f, pl.ds(j * LANES, LANES)],
                        sem,
                    )
                )
        for cp in copies:
            cp.wait()
        for f in range(NF):
            pltpu.sync_copy(
                buf_v.at[f],
                g_hbm.at[pl.ds(base_tok + c * CHUNK_TOK, CHUNK_TOK),
                         pl.ds(f * DIM, DIM)],
            )
        return carry

    lax.fori_loop(0, NCHUNK, chunk, 0)


_sc_gather = functools.partial(
    pl.kernel,
    mesh=plsc.VectorSubcoreMesh(core_axis_name="c", subcore_axis_name="s"),
    compiler_params=pltpu.CompilerParams(use_tc_tiling_on_sc=False),
    out_type=jax.ShapeDtypeStruct((N, NF * DIM), jnp.float32),
    scratch_types=[
        pltpu.VMEM((NF, ROWS_PER_W, LANES), jnp.int32),
        pltpu.VMEM((NF, CHUNK_TOK, DIM), jnp.float32),
        pltpu.SemaphoreType.DMA,
    ],
)(_sc_gather_body)


TILE = 1024


def _mlp_body(g_ref, ex_ref, w1m_ref, w1e_ref, b1_ref, w2_ref, b2_ref,
              w3_ref, b3_ref, p_ref, o_ref):
    hp = jnp.float32
    g = g_ref[...]
    ex = ex_ref[...]
    h = jnp.dot(g, w1m_ref[...], preferred_element_type=hp)
    h = h + jnp.dot(ex, w1e_ref[...], preferred_element_type=hp)
    h = jnp.maximum(h + b1_ref[...], 0.0)
    h = jnp.dot(h, w2_ref[...], preferred_element_type=hp) + b2_ref[...]
    h = jnp.maximum(h, 0.0)
    o = jnp.dot(h, w3_ref[...], preferred_element_type=hp) + b3_ref[...]
    o = o + jnp.dot(ex, p_ref[...], preferred_element_type=hp)
    o_ref[...] = o


def _mlp(g, ex, w1m, w1e, b1, w2, b2, w3e, b3e, p):
    grid = (N // TILE,)
    return pl.pallas_call(
        _mlp_body,
        grid=grid,
        in_specs=[
            pl.BlockSpec((TILE, NF * DIM), lambda i: (i, 0)),
            pl.BlockSpec((TILE, 3), lambda i: (i, 0)),
            pl.BlockSpec((NF * DIM, 128), lambda i: (0, 0)),
            pl.BlockSpec((3, 128), lambda i: (0, 0)),
            pl.BlockSpec((1, 128), lambda i: (0, 0)),
            pl.BlockSpec((128, 128), lambda i: (0, 0)),
            pl.BlockSpec((1, 128), lambda i: (0, 0)),
            pl.BlockSpec((128, OUT_DIM), lambda i: (0, 0)),
            pl.BlockSpec((1, OUT_DIM), lambda i: (0, 0)),
            pl.BlockSpec((3, OUT_DIM), lambda i: (0, 0)),
        ],
        out_specs=pl.BlockSpec((TILE, OUT_DIM), lambda i: (i, 0)),
        out_shape=jax.ShapeDtypeStruct((N, OUT_DIM), jnp.float32),
    )(g, ex, w1m, w1e, b1, w2, b2, w3e, b3e, p)


def kernel(x, table_0, table_1, table_2, table_3, W1, b1, W2, b2, W3, b3):
    xr = x.reshape(N, NF + 3)
    xi = xr[:, :NF].astype(jnp.int32)
    idxs = [xi[:, f].reshape(NW, ROWS_PER_W, LANES) for f in range(NF)]
    ex = xr[:, NF:]

    g = _sc_gather(table_0, table_1, table_2, table_3, *idxs)

    w1m = W1[: NF * DIM]
    w1e = W1[NF * DIM:]
    w3e = jnp.concatenate([W3, jnp.zeros((128, 1), jnp.float32)], axis=1)
    b3e = jnp.concatenate([b3, jnp.zeros((1,), jnp.float32)])[None, :]
    p = jnp.zeros((3, OUT_DIM), jnp.float32).at[2, OUT_DIM - 1].set(1.0)

    out = _mlp(g, ex, w1m, w1e, b1[None, :], W2, b2[None, :], w3e, b3e, p)
    return out.reshape(B, L, OUT_DIM)
